# Initial kernel scaffold; baseline (speedup 1.0000x reference)
import jax
import jax.numpy as jnp
from jax.experimental import pallas as pl


def kernel(x, edge_index, W1, b1, Wg, bg, W2, b2):
    raise NotImplementedError("write your pallas kernel here")


# speedup vs baseline: 27.2168x; 27.2168x over previous
import functools

import jax
import jax.numpy as jnp
from jax import lax
from jax.experimental import pallas as pl
from jax.experimental.pallas import tpu as pltpu
from jax.experimental.pallas import tpu_sc as plsc

N_REAL = 10000
D = 128
DH = D // 2
E_REAL = 320000

NC = 2
NS = 16
NW = NC * NS
CHUNK = 128
N_CHUNKS = 2560
E_P = N_CHUNKS * CHUNK
CHUNKS_PER_W = N_CHUNKS // NW
CHUNKS_PER_T = N_CHUNKS // NS
N_P = 10240
ROWS_PER_TILE = N_P // NS
BLK = 512

_mesh = plsc.VectorSubcoreMesh(core_axis_name="c", subcore_axis_name="s")


def _mm_nt(a, b):
    return lax.dot_general(a, b, (((1,), (1,)), ((), ())),
                           preferred_element_type=jnp.float32)


def _hw_body(x_ref, w1_ref, b1_ref, wg_ref, out_ref):
    h = jnp.maximum(_mm_nt(x_ref[...], w1_ref[...]) + b1_ref[...], 0.0)
    out_ref[...] = _mm_nt(h, wg_ref[...])


def _tc_hw(xp, W1, b1_2d, Wg):
    return pl.pallas_call(
        _hw_body,
        grid=(N_P // BLK,),
        in_specs=[
            pl.BlockSpec((BLK, D), lambda i: (i, 0)),
            pl.BlockSpec((D, D), lambda i: (0, 0)),
            pl.BlockSpec((1, D), lambda i: (0, 0)),
            pl.BlockSpec((D, D), lambda i: (0, 0)),
        ],
        out_specs=pl.BlockSpec((BLK, D), lambda i: (i, 0)),
        out_shape=jax.ShapeDtypeStruct((N_P, D), jnp.float32),
    )(xp, W1, b1_2d, Wg)


@functools.partial(
    pl.kernel,
    out_type=jax.ShapeDtypeStruct((NC, N_P), jnp.float32),
    mesh=_mesh,
    scratch_types=[
        pltpu.VMEM((CHUNKS_PER_W, CHUNK), jnp.int32),
        pltpu.VMEM((CHUNK,), jnp.float32),
        pltpu.VMEM((ROWS_PER_TILE,), jnp.float32),
        pltpu.VMEM_SHARED((N_P,), jnp.float32),
    ],
)
def _sc_degree(dst_hbm, out_hbm, idx_v, ones_v, buf_v, acc_sh):
    c = lax.axis_index("c")
    s = lax.axis_index("s")
    wid = s * NC + c

    def fill16(i, _):
        ones_v[pl.ds(i * 16, 16)] = jnp.ones((16,), jnp.float32)
        return 0
    lax.fori_loop(0, CHUNK // 16, fill16, 0)

    def zero16(i, _):
        buf_v[pl.ds(i * 16, 16)] = jnp.zeros((16,), jnp.float32)
        return 0
    lax.fori_loop(0, ROWS_PER_TILE // 16, zero16, 0)

    pltpu.sync_copy(buf_v, acc_sh.at[pl.ds(s * ROWS_PER_TILE, ROWS_PER_TILE)])
    plsc.subcore_barrier()

    pltpu.sync_copy(dst_hbm.at[pl.ds(wid * CHUNKS_PER_W, CHUNKS_PER_W)], idx_v)

    def body(j, _):
        pltpu.sync_copy(ones_v, acc_sh.at[idx_v.at[j]], add=True)
        return 0
    lax.fori_loop(0, CHUNKS_PER_W, body, 0)

    plsc.subcore_barrier()
    pltpu.sync_copy(acc_sh.at[pl.ds(s * ROWS_PER_TILE, ROWS_PER_TILE)], buf_v)
    pltpu.sync_copy(buf_v, out_hbm.at[c, pl.ds(s * ROWS_PER_TILE, ROWS_PER_TILE)])


def _scale_body(hw_ref, d0_ref, d1_ref, glo_ref, ghi_ref, dinv_ref):
    deg = d0_ref[...] + d1_ref[...] + 1.0
    dinv = lax.rsqrt(deg)
    dinv_ref[...] = dinv
    g = hw_ref[...] * dinv
    glo_ref[...] = g[:, :DH]
    ghi_ref[...] = g[:, DH:]


def _tc_scale(hw, deg0, deg1):
    return pl.pallas_call(
        _scale_body,
        grid=(N_P // BLK,),
        in_specs=[
            pl.BlockSpec((BLK, D), lambda i: (i, 0)),
            pl.BlockSpec((BLK, 1), lambda i: (i, 0)),
            pl.BlockSpec((BLK, 1), lambda i: (i, 0)),
        ],
        out_specs=[
            pl.BlockSpec((BLK, DH), lambda i: (i, 0)),
            pl.BlockSpec((BLK, DH), lambda i: (i, 0)),
            pl.BlockSpec((BLK, 1), lambda i: (i, 0)),
        ],
        out_shape=[
            jax.ShapeDtypeStruct((N_P, DH), jnp.float32),
            jax.ShapeDtypeStruct((N_P, DH), jnp.float32),
            jax.ShapeDtypeStruct((N_P, 1), jnp.float32),
        ],
    )(hw, deg0, deg1)


@functools.partial(
    pl.kernel,
    out_type=[
        jax.ShapeDtypeStruct((N_P, DH), jnp.float32),
        jax.ShapeDtypeStruct((N_P, DH), jnp.float32),
    ],
    mesh=_mesh,
    scratch_types=[
        pltpu.VMEM((CHUNKS_PER_T, CHUNK), jnp.int32),
        pltpu.VMEM((CHUNKS_PER_T, CHUNK), jnp.int32),
        pltpu.VMEM((CHUNK, DH), jnp.float32),
        pltpu.VMEM((CHUNK, DH), jnp.float32),
        pltpu.VMEM_SHARED((N_P, DH), jnp.float32),
        pltpu.SemaphoreType.DMA,
        pltpu.SemaphoreType.DMA,
    ],
    compiler_params=pltpu.CompilerParams(use_tc_tiling_on_sc=False),
)
def _sc_scatter(src_hbm, dst_hbm, glo_hbm, ghi_hbm, outlo_hbm, outhi_hbm,
                src_v, dst_v, bufa, bufb, acc_sh, sema, semb):
    c = lax.axis_index("c")
    s = lax.axis_index("s")

    def zero_row(i, _):
        bufa[i // (DH // 16), pl.ds((i % (DH // 16)) * 16, 16)] = (
            jnp.zeros((16,), jnp.float32))
        return 0
    lax.fori_loop(0, CHUNK * (DH // 16), zero_row, 0)

    def zero_acc(k, _):
        pltpu.sync_copy(bufa, acc_sh.at[pl.ds(s * ROWS_PER_TILE + k * CHUNK, CHUNK)])
        return 0
    lax.fori_loop(0, ROWS_PER_TILE // CHUNK, zero_acc, 0)
    plsc.subcore_barrier()

    pltpu.sync_copy(src_hbm.at[pl.ds(s * CHUNKS_PER_T, CHUNKS_PER_T)], src_v)
    pltpu.sync_copy(dst_hbm.at[pl.ds(s * CHUNKS_PER_T, CHUNKS_PER_T)], dst_v)

    def edge_loop(g_hbm):
        pltpu.async_copy(g_hbm.at[src_v.at[0]], bufa, sema)

        def body(jj, _):
            j0 = jj * 2
            pltpu.make_async_copy(g_hbm.at[src_v.at[j0]], bufa, sema).wait()
            pltpu.async_copy(g_hbm.at[src_v.at[j0 + 1]], bufb, semb)
            pltpu.sync_copy(bufa, acc_sh.at[dst_v.at[j0]], add=True)
            pltpu.make_async_copy(g_hbm.at[src_v.at[j0 + 1]], bufb, semb).wait()

            @pl.when(jj < CHUNKS_PER_T // 2 - 1)
            def _():
                pltpu.async_copy(g_hbm.at[src_v.at[j0 + 2]], bufa, sema)

            pltpu.sync_copy(bufb, acc_sh.at[dst_v.at[j0 + 1]], add=True)
            return 0
        lax.fori_loop(0, CHUNKS_PER_T // 2, body, 0)

    @pl.when(c == 0)
    def _():
        edge_loop(glo_hbm)

    @pl.when(c == 1)
    def _():
        edge_loop(ghi_hbm)

    plsc.subcore_barrier()

    def writeout(out_hbm):
        def wo(k, _):
            off = s * ROWS_PER_TILE + k * CHUNK
            pltpu.sync_copy(acc_sh.at[pl.ds(off, CHUNK)], bufb)
            pltpu.sync_copy(bufb, out_hbm.at[pl.ds(off, CHUNK)])
            return 0
        lax.fori_loop(0, ROWS_PER_TILE // CHUNK, wo, 0)

    @pl.when(c == 0)
    def _():
        writeout(outlo_hbm)

    @pl.when(c == 1)
    def _():
        writeout(outhi_hbm)


def _out_body(alo_ref, ahi_ref, glo_ref, ghi_ref, dinv_ref, bg_ref,
              w2_ref, b2_ref, o_ref):
    dinv = dinv_ref[...]
    conv = jnp.concatenate(
        [(alo_ref[...] + glo_ref[...]) * dinv,
         (ahi_ref[...] + ghi_ref[...]) * dinv], axis=1) + bg_ref[...]
    h2 = jnp.maximum(conv, 0.0)
    o_ref[...] = _mm_nt(h2, w2_ref[...]) + b2_ref[...]


def _tc_out(alo, ahi, glo, ghi, dinv, bg_2d, W2, b2_2d):
    return pl.pallas_call(
        _out_body,
        grid=(N_P // BLK,),
        in_specs=[
            pl.BlockSpec((BLK, DH), lambda i: (i, 0)),
            pl.BlockSpec((BLK, DH), lambda i: (i, 0)),
            pl.BlockSpec((BLK, DH), lambda i: (i, 0)),
            pl.BlockSpec((BLK, DH), lambda i: (i, 0)),
            pl.BlockSpec((BLK, 1), lambda i: (i, 0)),
            pl.BlockSpec((1, D), lambda i: (0, 0)),
            pl.BlockSpec((D, D), lambda i: (0, 0)),
            pl.BlockSpec((1, D), lambda i: (0, 0)),
        ],
        out_specs=pl.BlockSpec((BLK, D), lambda i: (i, 0)),
        out_shape=jax.ShapeDtypeStruct((N_P, D), jnp.float32),
    )(alo, ahi, glo, ghi, dinv, bg_2d, W2, b2_2d)


def kernel(x, edge_index, W1, b1, Wg, bg, W2, b2):
    ei = edge_index.astype(jnp.int32)
    n_pad_e = E_P - E_REAL
    pad_idx = N_REAL + (jnp.arange(n_pad_e, dtype=jnp.int32) % (N_P - N_REAL))
    srcp = jnp.concatenate([ei[0], pad_idx]).reshape(N_CHUNKS, CHUNK)
    dstp = jnp.concatenate([ei[1], pad_idx]).reshape(N_CHUNKS, CHUNK)

    xp = jnp.pad(x, ((0, N_P - N_REAL), (0, 0)))
    b1_2d = b1.reshape(1, D)
    bg_2d = bg.reshape(1, D)
    b2_2d = b2.reshape(1, D)

    hw = _tc_hw(xp, W1, b1_2d, Wg)

    deg_partials = _sc_degree(dstp)
    deg0 = deg_partials[0].reshape(N_P, 1)
    deg1 = deg_partials[1].reshape(N_P, 1)

    glo, ghi, dinv = _tc_scale(hw, deg0, deg1)

    alo, ahi = _sc_scatter(srcp, dstp, glo, ghi)

    out = _tc_out(alo, ahi, glo, ghi, dinv, bg_2d, W2, b2_2d)
    return out[:N_REAL]


# speedup vs baseline: 37.2207x; 1.3676x over previous
import functools

import jax
import jax.numpy as jnp
from jax import lax
from jax.experimental import pallas as pl
from jax.experimental.pallas import tpu as pltpu
from jax.experimental.pallas import tpu_sc as plsc

N_REAL = 10000
D = 128
DH = D // 2
E_REAL = 320000

NC = 2
NS = 16
NW = NC * NS
CHUNK = 128
N_CHUNKS = 2560
E_P = N_CHUNKS * CHUNK
CHUNKS_PER_W = N_CHUNKS // NW
CHUNKS_PER_T = N_CHUNKS // NS
N_P = 10240
ROWS_PER_TILE = N_P // NS
BLK = 512

_mesh = plsc.VectorSubcoreMesh(core_axis_name="c", subcore_axis_name="s")


def _mm_nt(a, b):
    return lax.dot_general(a, b, (((1,), (1,)), ((), ())),
                           preferred_element_type=jnp.float32)


def _hw_body(x_ref, w1_ref, b1_ref, wg_ref, out_ref):
    h = jnp.maximum(_mm_nt(x_ref[...], w1_ref[...]) + b1_ref[...], 0.0)
    out_ref[...] = _mm_nt(h, wg_ref[...])


def _tc_hw(xp, W1, b1_2d, Wg):
    return pl.pallas_call(
        _hw_body,
        grid=(N_P // BLK,),
        in_specs=[
            pl.BlockSpec((BLK, D), lambda i: (i, 0)),
            pl.BlockSpec((D, D), lambda i: (0, 0)),
            pl.BlockSpec((1, D), lambda i: (0, 0)),
            pl.BlockSpec((D, D), lambda i: (0, 0)),
        ],
        out_specs=pl.BlockSpec((BLK, D), lambda i: (i, 0)),
        out_shape=jax.ShapeDtypeStruct((N_P, D), jnp.float32),
    )(xp, W1, b1_2d, Wg)


@functools.partial(
    pl.kernel,
    out_type=jax.ShapeDtypeStruct((NC, N_P), jnp.float32),
    mesh=_mesh,
    scratch_types=[
        pltpu.VMEM((CHUNKS_PER_W, CHUNK), jnp.int32),
        pltpu.VMEM((CHUNK,), jnp.float32),
        pltpu.VMEM((ROWS_PER_TILE,), jnp.float32),
        pltpu.VMEM_SHARED((N_P,), jnp.float32),
    ],
)
def _sc_degree(dst_hbm, out_hbm, idx_v, ones_v, buf_v, acc_sh):
    c = lax.axis_index("c")
    s = lax.axis_index("s")
    wid = s * NC + c

    def fill16(i, _):
        ones_v[pl.ds(i * 16, 16)] = jnp.ones((16,), jnp.float32)
        return 0
    lax.fori_loop(0, CHUNK // 16, fill16, 0)

    def zero16(i, _):
        buf_v[pl.ds(i * 16, 16)] = jnp.zeros((16,), jnp.float32)
        return 0
    lax.fori_loop(0, ROWS_PER_TILE // 16, zero16, 0)

    pltpu.sync_copy(buf_v, acc_sh.at[pl.ds(s * ROWS_PER_TILE, ROWS_PER_TILE)])
    plsc.subcore_barrier()

    pltpu.sync_copy(dst_hbm.at[pl.ds(wid * CHUNKS_PER_W, CHUNKS_PER_W)], idx_v)

    def body(j, _):
        pltpu.sync_copy(ones_v, acc_sh.at[idx_v.at[j]], add=True)
        return 0
    lax.fori_loop(0, CHUNKS_PER_W, body, 0)

    plsc.subcore_barrier()
    pltpu.sync_copy(acc_sh.at[pl.ds(s * ROWS_PER_TILE, ROWS_PER_TILE)], buf_v)
    pltpu.sync_copy(buf_v, out_hbm.at[c, pl.ds(s * ROWS_PER_TILE, ROWS_PER_TILE)])


def _scale_body(hw_ref, d0_ref, d1_ref, glo_ref, ghi_ref, dinv_ref):
    deg = d0_ref[...] + d1_ref[...] + 1.0
    dinv = lax.rsqrt(deg)
    dinv_ref[...] = dinv
    g = hw_ref[...] * dinv
    glo_ref[...] = g[:, :DH]
    ghi_ref[...] = g[:, DH:]


def _tc_scale(hw, deg0, deg1):
    return pl.pallas_call(
        _scale_body,
        grid=(N_P // BLK,),
        in_specs=[
            pl.BlockSpec((BLK, D), lambda i: (i, 0)),
            pl.BlockSpec((BLK, 1), lambda i: (i, 0)),
            pl.BlockSpec((BLK, 1), lambda i: (i, 0)),
        ],
        out_specs=[
            pl.BlockSpec((BLK, DH), lambda i: (i, 0)),
            pl.BlockSpec((BLK, DH), lambda i: (i, 0)),
            pl.BlockSpec((BLK, 1), lambda i: (i, 0)),
        ],
        out_shape=[
            jax.ShapeDtypeStruct((N_P, DH), jnp.float32),
            jax.ShapeDtypeStruct((N_P, DH), jnp.float32),
            jax.ShapeDtypeStruct((N_P, 1), jnp.float32),
        ],
    )(hw, deg0, deg1)


@functools.partial(
    pl.kernel,
    out_type=[
        jax.ShapeDtypeStruct((N_P, DH), jnp.float32),
        jax.ShapeDtypeStruct((N_P, DH), jnp.float32),
    ],
    mesh=_mesh,
    scratch_types=[
        pltpu.VMEM((CHUNKS_PER_T, CHUNK), jnp.int32),
        pltpu.VMEM((CHUNKS_PER_T, CHUNK), jnp.int32),
        pltpu.VMEM((4, CHUNK, DH), jnp.float32),
        pltpu.VMEM_SHARED((N_P, DH), jnp.float32),
        [pltpu.SemaphoreType.DMA] * 4,
        [pltpu.SemaphoreType.DMA] * 4,
    ],
    compiler_params=pltpu.CompilerParams(use_tc_tiling_on_sc=False),
)
def _sc_scatter(src_hbm, dst_hbm, glo_hbm, ghi_hbm, outlo_hbm, outhi_hbm,
                src_v, dst_v, ring, acc_sh, gsems, ssems):
    c = lax.axis_index("c")
    s = lax.axis_index("s")

    def zero_row(i, _):
        ring[0, i // (DH // 16), pl.ds((i % (DH // 16)) * 16, 16)] = (
            jnp.zeros((16,), jnp.float32))
        return 0
    lax.fori_loop(0, CHUNK * (DH // 16), zero_row, 0)

    def zero_acc(k, _):
        pltpu.sync_copy(ring.at[0],
                        acc_sh.at[pl.ds(s * ROWS_PER_TILE + k * CHUNK, CHUNK)])
        return 0
    lax.fori_loop(0, ROWS_PER_TILE // CHUNK, zero_acc, 0)
    plsc.subcore_barrier()

    pltpu.sync_copy(src_hbm.at[pl.ds(s * CHUNKS_PER_T, CHUNKS_PER_T)], src_v)
    pltpu.sync_copy(dst_hbm.at[pl.ds(s * CHUNKS_PER_T, CHUNKS_PER_T)], dst_v)

    def gather(g_hbm):
        for k in range(3):
            pltpu.async_copy(g_hbm.at[src_v.at[k]], ring.at[k], gsems[k])

        def body(jj, _):
            for k in range(4):
                j = jj * 4 + k
                m = (k + 3) % 4
                pltpu.make_async_copy(g_hbm.at[src_v.at[j]], ring.at[k],
                                      gsems[k]).wait()

                @pl.when(jnp.logical_and(j >= 1, j + 3 < CHUNKS_PER_T))
                def _():
                    pltpu.make_async_copy(ring.at[m],
                                          acc_sh.at[dst_v.at[j]],
                                          ssems[m]).wait()

                @pl.when(j + 3 < CHUNKS_PER_T)
                def _():
                    pltpu.async_copy(g_hbm.at[src_v.at[j + 3]], ring.at[m],
                                     gsems[m])

                pltpu.async_copy(ring.at[k], acc_sh.at[dst_v.at[j]],
                                 ssems[k], add=True)
            return 0
        lax.fori_loop(0, CHUNKS_PER_T // 4, body, 0)

        for k in range(4):
            pltpu.make_async_copy(ring.at[k], acc_sh.at[dst_v.at[0]],
                                  ssems[k]).wait()

    @pl.when(c == 0)
    def _():
        gather(glo_hbm)

    @pl.when(c == 1)
    def _():
        gather(ghi_hbm)

    plsc.subcore_barrier()

    def writeout(out_hbm):
        def wo(k, _):
            off = s * ROWS_PER_TILE + k * CHUNK
            pltpu.sync_copy(acc_sh.at[pl.ds(off, CHUNK)], ring.at[0])
            pltpu.sync_copy(ring.at[0], out_hbm.at[pl.ds(off, CHUNK)])
            return 0
        lax.fori_loop(0, ROWS_PER_TILE // CHUNK, wo, 0)

    @pl.when(c == 0)
    def _():
        writeout(outlo_hbm)

    @pl.when(c == 1)
    def _():
        writeout(outhi_hbm)


def _out_body(alo_ref, ahi_ref, glo_ref, ghi_ref, dinv_ref, bg_ref,
              w2_ref, b2_ref, o_ref):
    dinv = dinv_ref[...]
    conv = jnp.concatenate(
        [(alo_ref[...] + glo_ref[...]) * dinv,
         (ahi_ref[...] + ghi_ref[...]) * dinv], axis=1) + bg_ref[...]
    h2 = jnp.maximum(conv, 0.0)
    o_ref[...] = _mm_nt(h2, w2_ref[...]) + b2_ref[...]


def _tc_out(alo, ahi, glo, ghi, dinv, bg_2d, W2, b2_2d):
    return pl.pallas_call(
        _out_body,
        grid=(N_P // BLK,),
        in_specs=[
            pl.BlockSpec((BLK, DH), lambda i: (i, 0)),
            pl.BlockSpec((BLK, DH), lambda i: (i, 0)),
            pl.BlockSpec((BLK, DH), lambda i: (i, 0)),
            pl.BlockSpec((BLK, DH), lambda i: (i, 0)),
            pl.BlockSpec((BLK, 1), lambda i: (i, 0)),
            pl.BlockSpec((1, D), lambda i: (0, 0)),
            pl.BlockSpec((D, D), lambda i: (0, 0)),
            pl.BlockSpec((1, D), lambda i: (0, 0)),
        ],
        out_specs=pl.BlockSpec((BLK, D), lambda i: (i, 0)),
        out_shape=jax.ShapeDtypeStruct((N_P, D), jnp.float32),
    )(alo, ahi, glo, ghi, dinv, bg_2d, W2, b2_2d)


def kernel(x, edge_index, W1, b1, Wg, bg, W2, b2):
    ei = edge_index.astype(jnp.int32)
    n_pad_e = E_P - E_REAL
    pad_idx = N_REAL + (jnp.arange(n_pad_e, dtype=jnp.int32) % (N_P - N_REAL))
    srcp = jnp.concatenate([ei[0], pad_idx]).reshape(N_CHUNKS, CHUNK)
    dstp = jnp.concatenate([ei[1], pad_idx]).reshape(N_CHUNKS, CHUNK)

    xp = jnp.pad(x, ((0, N_P - N_REAL), (0, 0)))
    b1_2d = b1.reshape(1, D)
    bg_2d = bg.reshape(1, D)
    b2_2d = b2.reshape(1, D)

    hw = _tc_hw(xp, W1, b1_2d, Wg)

    deg_partials = _sc_degree(dstp)
    deg0 = deg_partials[0].reshape(N_P, 1)
    deg1 = deg_partials[1].reshape(N_P, 1)

    glo, ghi, dinv = _tc_scale(hw, deg0, deg1)

    alo, ahi = _sc_scatter(srcp, dstp, glo, ghi)

    out = _tc_out(alo, ahi, glo, ghi, dinv, bg_2d, W2, b2_2d)
    return out[:N_REAL]
